# baseline (speedup 1.0000x reference)
import functools

import jax
import jax.numpy as jnp
from jax import lax
from jax.experimental import pallas as pl
from jax.experimental.pallas import tpu as pltpu
from jax.experimental.pallas import tpu_sc as plsc

B, N, S, C, D = 2, 8192, 4096, 3, 128
K = 16
SB = 256
LANES = 16


def _topk_tc_body(xyz_ref, sxyz_ref, sxyzT_ref, idx_ref):
    q = xyz_ref[0]
    s = sxyz_ref[0]
    sT = sxyzT_ref[0]

    dot = lax.dot_general(
        q, sT, (((1,), (0,)), ((), ())),
        preferred_element_type=jnp.float32)
    qn = jnp.sum(q * q, axis=1, keepdims=True)
    sn = jnp.sum(s * s, axis=1).reshape(1, N)
    dist = -2.0 * dot
    dist = dist + qn
    dist = dist + sn
    neg0 = -dist

    iota_f = lax.broadcasted_iota(
        jnp.int32, (SB, N), 1).astype(jnp.float32)
    kcol16 = lax.broadcasted_iota(jnp.int32, (SB, K), 1)
    big = jnp.float32(N)

    def step(k, carry):
        i_prev, neg, idx_acc = carry
        neg = jnp.where(iota_f == i_prev[:, None], -jnp.inf, neg)
        m = jnp.max(neg, axis=1)
        i_f = jnp.min(jnp.where(neg == m[:, None], iota_f, big),
                      axis=1)
        idx_acc = jnp.where(kcol16 == k, i_f.astype(jnp.int32)[:, None],
                            idx_acc)
        return i_f, neg, idx_acc

    _, _, idx_acc = lax.fori_loop(
        0, K, step,
        (jnp.full((SB,), -1.0, jnp.float32),
         neg0,
         jnp.zeros((SB, K), jnp.int32)))
    idx_ref[0] = idx_acc


def _topk_tc(xyz_b, s_xyz_b, s_xyzT_b):
    grid = (S // SB,)
    return pl.pallas_call(
        _topk_tc_body,
        grid=grid,
        in_specs=[
            pl.BlockSpec((1, SB, C), lambda j: (0, j, 0)),
            pl.BlockSpec((1, N, C), lambda j: (0, 0, 0)),
            pl.BlockSpec((1, C, N), lambda j: (0, 0, 0)),
        ],
        out_specs=pl.BlockSpec((1, SB, K), lambda j: (0, j, 0)),
        out_shape=jax.ShapeDtypeStruct((1, S, K), jnp.int32),
    )(xyz_b, s_xyz_b, s_xyzT_b)


_NC, _NS = 2, 16
_NW = _NC * _NS
_RB = S * K
_RW = _RB // _NW
_QW = _RW // K
_CHUNK = 128
_GRP = _CHUNK // LANES


def _gather_sc_body(points_hbm, xyzpad_hbm, idx_hbm, qp_hbm,
                    feat_hbm, xyz_hbm,
                    idx_v, rows_f, rows_x, q_v, xyz_o, semf, semx):
    wid = lax.axis_index("s") * _NC + lax.axis_index("c")
    base = wid * _RW
    pltpu.sync_copy(idx_hbm.at[pl.ds(base, _RW)], idx_v)
    pltpu.sync_copy(qp_hbm.at[pl.ds(wid * _QW, _QW)], q_v)

    def chunk(c, carry):
        off = c * _CHUNK
        idx_slice = idx_v.at[pl.ds(off, _CHUNK)]
        cp_f = pltpu.async_copy(points_hbm.at[idx_slice], rows_f, semf)
        cp_x = pltpu.async_copy(xyzpad_hbm.at[idx_slice], rows_x, semx)
        cp_x.wait()
        qbase = c * _GRP
        for r in range(_CHUNK):
            xyz_o[r] = rows_x[r, pl.ds(0, LANES)] - q_v[qbase + r // K]
        cp_f.wait()
        pltpu.sync_copy(rows_f, feat_hbm.at[pl.ds(base + off, _CHUNK)])
        pltpu.sync_copy(xyz_o, xyz_hbm.at[pl.ds(base + off, _CHUNK)])
        return carry

    lax.fori_loop(0, _RW // _CHUNK, chunk, 0)


@functools.lru_cache(maxsize=None)
def _gather_sc():
    return pl.kernel(
        _gather_sc_body,
        mesh=plsc.VectorSubcoreMesh(core_axis_name="c", subcore_axis_name="s"),
        out_type=[
            jax.ShapeDtypeStruct((_RB, D), jnp.float32),
            jax.ShapeDtypeStruct((_RB, LANES), jnp.float32),
        ],
        scratch_types=[
            pltpu.VMEM((_RW,), jnp.int32),
            pltpu.VMEM((_CHUNK, D), jnp.float32),
            pltpu.VMEM((_CHUNK, D), jnp.float32),
            pltpu.VMEM((_QW, LANES), jnp.float32),
            pltpu.VMEM((_CHUNK, LANES), jnp.float32),
            pltpu.SemaphoreType.DMA,
            pltpu.SemaphoreType.DMA,
        ],
    )


def kernel(s_xyz, xyz, s_points, nsample):
    s_xyzT = jnp.swapaxes(s_xyz, 1, 2)
    feats, xyzs = [], []
    for b in range(B):
        idx_b = _topk_tc(xyz[b:b + 1], s_xyz[b:b + 1], s_xyzT[b:b + 1])
        xyzpad_b = jnp.pad(s_xyz[b], ((0, 0), (0, D - C)))
        qpad_b = jnp.pad(xyz[b], ((0, 0), (0, LANES - C)))
        feat_b, xyz_b = _gather_sc()(
            s_points[b], xyzpad_b, idx_b.reshape(_RB), qpad_b)
        feats.append(feat_b)
        xyzs.append(xyz_b)
    feat = jnp.concatenate(feats, axis=0)
    gx = jnp.concatenate(xyzs, axis=0)[:, :C]
    new_points = jnp.concatenate([gx, feat], axis=1).reshape(B, S, K, C + D)
    grouped_xyz_norm = new_points[..., :C]
    return new_points, grouped_xyz_norm


# speedup vs baseline: 1.0384x; 1.0384x over previous
import functools

import jax
import jax.numpy as jnp
from jax import lax
from jax.experimental import pallas as pl
from jax.experimental.pallas import tpu as pltpu
from jax.experimental.pallas import tpu_sc as plsc

B, N, S, C, D = 2, 8192, 4096, 3, 128
K = 16
SB = 512
LANES = 16


def _topk_tc_body(xyz_ref, sxyz_ref, sxyzT_ref, idx_ref):
    q = xyz_ref[0]
    s = sxyz_ref[0]
    sT = sxyzT_ref[0]

    dot = lax.dot_general(
        q, sT, (((1,), (0,)), ((), ())),
        preferred_element_type=jnp.float32)
    qn = jnp.sum(q * q, axis=1, keepdims=True)
    sn = jnp.sum(s * s, axis=1).reshape(1, N)
    neg0 = 2.0 * dot
    neg0 = neg0 - qn
    neg0 = neg0 - sn

    iota_f = lax.broadcasted_iota(
        jnp.int32, (1, N), 1).astype(jnp.float32)
    kcol16 = lax.broadcasted_iota(jnp.int32, (SB, K), 1)
    big = jnp.float32(N)

    def step(k, carry):
        i_prev, neg, idx_acc = carry
        neg = jnp.where(iota_f == i_prev[:, None], -jnp.inf, neg)
        m = jnp.max(neg, axis=1)
        i_f = jnp.min(jnp.where(neg == m[:, None], iota_f, big),
                      axis=1)
        idx_acc = jnp.where(kcol16 == k, i_f.astype(jnp.int32)[:, None],
                            idx_acc)
        return i_f, neg, idx_acc

    _, _, idx_acc = lax.fori_loop(
        0, K, step,
        (jnp.full((SB,), -1.0, jnp.float32),
         neg0,
         jnp.zeros((SB, K), jnp.int32)))
    idx_ref[0] = idx_acc


def _topk_tc(xyz_b, s_xyz_b, s_xyzT_b):
    grid = (S // SB,)
    return pl.pallas_call(
        _topk_tc_body,
        grid=grid,
        in_specs=[
            pl.BlockSpec((1, SB, C), lambda j: (0, j, 0)),
            pl.BlockSpec((1, N, C), lambda j: (0, 0, 0)),
            pl.BlockSpec((1, C, N), lambda j: (0, 0, 0)),
        ],
        out_specs=pl.BlockSpec((1, SB, K), lambda j: (0, j, 0)),
        out_shape=jax.ShapeDtypeStruct((1, S, K), jnp.int32),
    )(xyz_b, s_xyz_b, s_xyzT_b)


_NC, _NS = 2, 16
_NW = _NC * _NS
_RB = S * K
_RW = _RB // _NW
_QW = _RW // K
_CHUNK = 128
_GRP = _CHUNK // LANES


def _gather_sc_body(points_hbm, xyzpad_hbm, idx_hbm, qp_hbm,
                    feat_hbm, xyz_hbm,
                    idx_v, rows_f, rows_x, q_v, xyz_o, semf, semx):
    wid = lax.axis_index("s") * _NC + lax.axis_index("c")
    base = wid * _RW
    pltpu.sync_copy(idx_hbm.at[pl.ds(base, _RW)], idx_v)
    pltpu.sync_copy(qp_hbm.at[pl.ds(wid * _QW, _QW)], q_v)

    def chunk(c, carry):
        off = c * _CHUNK
        idx_slice = idx_v.at[pl.ds(off, _CHUNK)]
        cp_f = pltpu.async_copy(points_hbm.at[idx_slice], rows_f, semf)
        cp_x = pltpu.async_copy(xyzpad_hbm.at[idx_slice], rows_x, semx)
        cp_x.wait()
        qbase = c * _GRP
        for r in range(_CHUNK):
            xyz_o[r] = rows_x[r, pl.ds(0, LANES)] - q_v[qbase + r // K]
        cp_f.wait()
        pltpu.sync_copy(rows_f, feat_hbm.at[pl.ds(base + off, _CHUNK)])
        pltpu.sync_copy(xyz_o, xyz_hbm.at[pl.ds(base + off, _CHUNK)])
        return carry

    lax.fori_loop(0, _RW // _CHUNK, chunk, 0)


@functools.lru_cache(maxsize=None)
def _gather_sc():
    return pl.kernel(
        _gather_sc_body,
        mesh=plsc.VectorSubcoreMesh(core_axis_name="c", subcore_axis_name="s"),
        out_type=[
            jax.ShapeDtypeStruct((_RB, D), jnp.float32),
            jax.ShapeDtypeStruct((_RB, LANES), jnp.float32),
        ],
        scratch_types=[
            pltpu.VMEM((_RW,), jnp.int32),
            pltpu.VMEM((_CHUNK, D), jnp.float32),
            pltpu.VMEM((_CHUNK, D), jnp.float32),
            pltpu.VMEM((_QW, LANES), jnp.float32),
            pltpu.VMEM((_CHUNK, LANES), jnp.float32),
            pltpu.SemaphoreType.DMA,
            pltpu.SemaphoreType.DMA,
        ],
    )


def kernel(s_xyz, xyz, s_points, nsample):
    s_xyzT = jnp.swapaxes(s_xyz, 1, 2)
    feats, xyzs = [], []
    for b in range(B):
        idx_b = _topk_tc(xyz[b:b + 1], s_xyz[b:b + 1], s_xyzT[b:b + 1])
        xyzpad_b = jnp.pad(s_xyz[b], ((0, 0), (0, D - C)))
        qpad_b = jnp.pad(xyz[b], ((0, 0), (0, LANES - C)))
        feat_b, xyz_b = _gather_sc()(
            s_points[b], xyzpad_b, idx_b.reshape(_RB), qpad_b)
        feats.append(feat_b)
        xyzs.append(xyz_b)
    feat = jnp.concatenate(feats, axis=0)
    gx = jnp.concatenate(xyzs, axis=0)[:, :C]
    new_points = jnp.concatenate([gx, feat], axis=1).reshape(B, S, K, C + D)
    grouped_xyz_norm = new_points[..., :C]
    return new_points, grouped_xyz_norm


# speedup vs baseline: 1.0417x; 1.0032x over previous
import functools

import jax
import jax.numpy as jnp
from jax import lax
from jax.experimental import pallas as pl
from jax.experimental.pallas import tpu as pltpu
from jax.experimental.pallas import tpu_sc as plsc

B, N, S, C, D = 2, 8192, 4096, 3, 128
K = 16
SB = 512
LANES = 16


def _topk_tc_body(xyz_ref, sxyz_ref, sxyzT_ref, idx_ref):
    q = xyz_ref[0]
    s = sxyz_ref[0]
    sT = sxyzT_ref[0]

    dot = lax.dot_general(
        q, sT, (((1,), (0,)), ((), ())),
        preferred_element_type=jnp.float32)
    qn = jnp.sum(q * q, axis=1, keepdims=True)
    sn = jnp.sum(s * s, axis=1).reshape(1, N)
    neg0 = 2.0 * dot
    neg0 = neg0 - qn
    neg0 = neg0 - sn

    iota_f = lax.broadcasted_iota(
        jnp.int32, (1, N), 1).astype(jnp.float32)
    kcol16 = lax.broadcasted_iota(jnp.int32, (SB, K), 1)
    big = jnp.float32(N)

    b = pl.program_id(0)

    def step(k, carry):
        i_prev, neg, idx_acc = carry
        neg = jnp.where(iota_f == i_prev[:, None], -jnp.inf, neg)
        m = jnp.max(neg, axis=1)
        i_f = jnp.min(jnp.where(neg == m[:, None], iota_f, big),
                      axis=1)
        idx_acc = jnp.where(kcol16 == k,
                            (i_f.astype(jnp.int32) + b * N)[:, None],
                            idx_acc)
        return i_f, neg, idx_acc

    _, _, idx_acc = lax.fori_loop(
        0, K, step,
        (jnp.full((SB,), -1.0, jnp.float32),
         neg0,
         jnp.zeros((SB, K), jnp.int32)))
    idx_ref[0] = idx_acc


def _topk_tc(xyz, s_xyz, s_xyzT):
    grid = (B, S // SB)
    return pl.pallas_call(
        _topk_tc_body,
        grid=grid,
        in_specs=[
            pl.BlockSpec((1, SB, C), lambda b, j: (b, j, 0)),
            pl.BlockSpec((1, N, C), lambda b, j: (b, 0, 0)),
            pl.BlockSpec((1, C, N), lambda b, j: (b, 0, 0)),
        ],
        out_specs=pl.BlockSpec((1, SB, K), lambda b, j: (b, j, 0)),
        out_shape=jax.ShapeDtypeStruct((B, S, K), jnp.int32),
    )(xyz, s_xyz, s_xyzT)


_NC, _NS = 2, 16
_NW = _NC * _NS
_RB = B * S * K
_RW = _RB // _NW
_QW = _RW // K
_CHUNK = 128
_GRP = _CHUNK // LANES


def _gather_sc_body(points_hbm, xyzpad_hbm, idx_hbm, qp_hbm,
                    feat_hbm, xyz_hbm,
                    idx_v, rows_f, rows_x, q_v, xyz_o, semf, semx):
    wid = lax.axis_index("s") * _NC + lax.axis_index("c")
    base = wid * _RW
    pltpu.sync_copy(idx_hbm.at[pl.ds(base, _RW)], idx_v)
    pltpu.sync_copy(qp_hbm.at[pl.ds(wid * _QW, _QW)], q_v)

    def chunk(c, carry):
        off = c * _CHUNK
        idx_slice = idx_v.at[pl.ds(off, _CHUNK)]
        cp_f = pltpu.async_copy(points_hbm.at[idx_slice], rows_f, semf)
        cp_x = pltpu.async_copy(xyzpad_hbm.at[idx_slice], rows_x, semx)
        cp_x.wait()
        qbase = c * _GRP
        for r in range(_CHUNK):
            xyz_o[r] = rows_x[r, pl.ds(0, LANES)] - q_v[qbase + r // K]
        cp_f.wait()
        pltpu.sync_copy(rows_f, feat_hbm.at[pl.ds(base + off, _CHUNK)])
        pltpu.sync_copy(xyz_o, xyz_hbm.at[pl.ds(base + off, _CHUNK)])
        return carry

    lax.fori_loop(0, _RW // _CHUNK, chunk, 0)


@functools.lru_cache(maxsize=None)
def _gather_sc():
    return pl.kernel(
        _gather_sc_body,
        mesh=plsc.VectorSubcoreMesh(core_axis_name="c", subcore_axis_name="s"),
        out_type=[
            jax.ShapeDtypeStruct((_RB, D), jnp.float32),
            jax.ShapeDtypeStruct((_RB, LANES), jnp.float32),
        ],
        scratch_types=[
            pltpu.VMEM((_RW,), jnp.int32),
            pltpu.VMEM((_CHUNK, D), jnp.float32),
            pltpu.VMEM((_CHUNK, D), jnp.float32),
            pltpu.VMEM((_QW, LANES), jnp.float32),
            pltpu.VMEM((_CHUNK, LANES), jnp.float32),
            pltpu.SemaphoreType.DMA,
            pltpu.SemaphoreType.DMA,
        ],
    )


def kernel(s_xyz, xyz, s_points, nsample):
    s_xyzT = jnp.swapaxes(s_xyz, 1, 2)
    idx = _topk_tc(xyz, s_xyz, s_xyzT)
    xyzpad = jnp.pad(s_xyz.reshape(B * N, C), ((0, 0), (0, D - C)))
    qpad = jnp.pad(xyz.reshape(B * S, C), ((0, 0), (0, LANES - C)))
    feat, gx16 = _gather_sc()(
        s_points.reshape(B * N, D), xyzpad, idx.reshape(_RB), qpad)
    new_points = jnp.concatenate(
        [gx16[:, :C], feat], axis=1).reshape(B, S, K, C + D)
    grouped_xyz_norm = new_points[..., :C]
    return new_points, grouped_xyz_norm


# speedup vs baseline: 1.0583x; 1.0159x over previous
import functools

import jax
import jax.numpy as jnp
from jax import lax
from jax.experimental import pallas as pl
from jax.experimental.pallas import tpu as pltpu
from jax.experimental.pallas import tpu_sc as plsc

B, N, S, C, D = 2, 8192, 4096, 3, 128
K = 16
SB = 512
LANES = 16


def _topk_tc_body(xyz_ref, sxyz_ref, sxyzT_ref, idx_ref):
    q = xyz_ref[0]
    s = sxyz_ref[0]
    sT = sxyzT_ref[0]

    dot = lax.dot_general(
        q, sT, (((1,), (0,)), ((), ())),
        preferred_element_type=jnp.float32)
    qn = jnp.sum(q * q, axis=1, keepdims=True)
    sn = jnp.sum(s * s, axis=1).reshape(1, N)
    neg0 = 2.0 * dot
    neg0 = neg0 - qn
    neg0 = neg0 - sn

    iota_f = lax.broadcasted_iota(
        jnp.int32, (1, N), 1).astype(jnp.float32)
    kcol16 = lax.broadcasted_iota(jnp.int32, (SB, K), 1)
    big = jnp.float32(N)

    b = pl.program_id(0)

    def step(k, carry):
        i_prev, neg, idx_acc = carry
        neg = jnp.where(iota_f == i_prev[:, None], -jnp.inf, neg)
        m = jnp.max(neg, axis=1)
        i_f = jnp.min(jnp.where(neg == m[:, None], iota_f, big),
                      axis=1)
        idx_acc = jnp.where(kcol16 == k,
                            (i_f.astype(jnp.int32) + b * N)[:, None],
                            idx_acc)
        return i_f, neg, idx_acc

    _, _, idx_acc = lax.fori_loop(
        0, K, step,
        (jnp.full((SB,), -1.0, jnp.float32),
         neg0,
         jnp.zeros((SB, K), jnp.int32)))
    idx_ref[0] = idx_acc


def _topk_tc(xyz, s_xyz, s_xyzT):
    grid = (B, S // SB)
    return pl.pallas_call(
        _topk_tc_body,
        grid=grid,
        in_specs=[
            pl.BlockSpec((1, SB, C), lambda b, j: (b, j, 0)),
            pl.BlockSpec((1, N, C), lambda b, j: (b, 0, 0)),
            pl.BlockSpec((1, C, N), lambda b, j: (b, 0, 0)),
        ],
        out_specs=pl.BlockSpec((1, SB, K), lambda b, j: (b, j, 0)),
        out_shape=jax.ShapeDtypeStruct((B, S, K), jnp.int32),
    )(xyz, s_xyz, s_xyzT)


_NC, _NS = 2, 16
_NW = _NC * _NS
_RB = B * S * K
_RW = _RB // _NW
_QW = _RW // K
_CHUNK = 128
_GRP = _CHUNK // LANES


_NCH = _RW // _CHUNK
_NBUF = 2


def _gather_sc_body(points_hbm, xyzpad_hbm, idx_hbm, qp_hbm,
                    feat_hbm, xyz_hbm,
                    idx_v, rows_f0, rows_f1, rows_x0, rows_x1,
                    q_v, xyz_o, semf0, semf1, semx0, semx1):
    wid = lax.axis_index("s") * _NC + lax.axis_index("c")
    base = wid * _RW
    pltpu.sync_copy(idx_hbm.at[pl.ds(base, _RW)], idx_v)
    pltpu.sync_copy(qp_hbm.at[pl.ds(wid * _QW, _QW)], q_v)

    rows_f = (rows_f0, rows_f1)
    rows_x = (rows_x0, rows_x1)
    semf = (semf0, semf1)
    semx = (semx0, semx1)

    def fire(c, bi):
        idx_slice = idx_v.at[pl.ds(c * _CHUNK, _CHUNK)]
        pltpu.async_copy(points_hbm.at[idx_slice], rows_f[bi], semf[bi])
        pltpu.async_copy(xyzpad_hbm.at[idx_slice], rows_x[bi], semx[bi])

    def drain(bi):
        pltpu.make_async_copy(points_hbm.at[pl.ds(0, _CHUNK)],
                              rows_f[bi], semf[bi]).wait()
        pltpu.make_async_copy(xyzpad_hbm.at[pl.ds(0, _CHUNK)],
                              rows_x[bi], semx[bi]).wait()

    for bi in range(_NBUF):
        fire(bi, bi)

    def pair(cc, carry):
        for bi in range(_NBUF):
            c = cc * _NBUF + bi
            off = c * _CHUNK
            drain(bi)
            qbase = c * _GRP
            for r in range(_CHUNK):
                xyz_o[r] = rows_x[bi][r, pl.ds(0, LANES)] - q_v[qbase + r // K]
            pltpu.sync_copy(rows_f[bi], feat_hbm.at[pl.ds(base + off, _CHUNK)])
            pltpu.sync_copy(xyz_o, xyz_hbm.at[pl.ds(base + off, _CHUNK)])

            @pl.when(c + _NBUF < _NCH)
            def _():
                fire(c + _NBUF, bi)
        return carry

    lax.fori_loop(0, _NCH // _NBUF, pair, 0)


@functools.lru_cache(maxsize=None)
def _gather_sc():
    return pl.kernel(
        _gather_sc_body,
        mesh=plsc.VectorSubcoreMesh(core_axis_name="c", subcore_axis_name="s"),
        out_type=[
            jax.ShapeDtypeStruct((_RB, D), jnp.float32),
            jax.ShapeDtypeStruct((_RB, LANES), jnp.float32),
        ],
        scratch_types=[
            pltpu.VMEM((_RW,), jnp.int32),
            pltpu.VMEM((_CHUNK, D), jnp.float32),
            pltpu.VMEM((_CHUNK, D), jnp.float32),
            pltpu.VMEM((_CHUNK, D), jnp.float32),
            pltpu.VMEM((_CHUNK, D), jnp.float32),
            pltpu.VMEM((_QW, LANES), jnp.float32),
            pltpu.VMEM((_CHUNK, LANES), jnp.float32),
            pltpu.SemaphoreType.DMA,
            pltpu.SemaphoreType.DMA,
            pltpu.SemaphoreType.DMA,
            pltpu.SemaphoreType.DMA,
        ],
    )


def kernel(s_xyz, xyz, s_points, nsample):
    s_xyzT = jnp.swapaxes(s_xyz, 1, 2)
    idx = _topk_tc(xyz, s_xyz, s_xyzT)
    xyzpad = jnp.pad(s_xyz.reshape(B * N, C), ((0, 0), (0, D - C)))
    qpad = jnp.pad(xyz.reshape(B * S, C), ((0, 0), (0, LANES - C)))
    feat, gx16 = _gather_sc()(
        s_points.reshape(B * N, D), xyzpad, idx.reshape(_RB), qpad)
    new_points = jnp.concatenate(
        [gx16[:, :C], feat], axis=1).reshape(B, S, K, C + D)
    grouped_xyz_norm = new_points[..., :C]
    return new_points, grouped_xyz_norm
